# baseline (speedup 1.0000x reference)
import functools

import jax
import jax.numpy as jnp
from jax import lax
from jax.experimental import pallas as pl
from jax.experimental.pallas import tpu as pltpu
from jax.experimental.pallas import tpu_sc as plsc

NUM_EMB = 1000
SEM_IDS_DIM = 4
EMB_DIM = 128
PAD_IDX = SEM_IDS_DIM * NUM_EMB
B, L = 4096, 200
NSEQ = B * L
NFUT = B * SEM_IDS_DIM

NC, NS = 2, 16
NW = NC * NS
CHUNK = 256
GATHER = 128
SEQ_PER_W = NSEQ // NW
FUT_PER_W = NFUT // NW
SEQ_CHUNKS = SEQ_PER_W // CHUNK
FUT_CHUNKS = FUT_PER_W // CHUNK


def _body(tt, sem, msk, ttf, semf, emb, seq_out, fut_out,
          tt_v, sem_v, msk_v, idx_v, rows_v, gsem):
    wid = lax.axis_index("s") * NC + lax.axis_index("c")

    def do_chunk(src_tt, src_sem, src_msk, dst, base):
        pltpu.sync_copy(src_tt.at[pl.ds(base, CHUNK)], tt_v)
        pltpu.sync_copy(src_sem.at[pl.ds(base, CHUNK)], sem_v)
        if src_msk is not None:
            pltpu.sync_copy(src_msk.at[pl.ds(base, CHUNK)], msk_v)
        for i in range(CHUNK // 16):
            t = tt_v[pl.ds(i * 16, 16)]
            s = sem_v[pl.ds(i * 16, 16)]
            ids = t * NUM_EMB + s
            if src_msk is not None:
                m = msk_v[pl.ds(i * 16, 16)]
                ids = jnp.where(m != 0, ids, PAD_IDX)
            idx_v[pl.ds(i * 16, 16)] = ids
        copies = [
            pltpu.async_copy(emb.at[idx_v.at[pl.ds(j * GATHER, GATHER)]],
                             rows_v.at[pl.ds(j * GATHER, GATHER)], gsem)
            for j in range(CHUNK // GATHER)
        ]
        for c in copies:
            c.wait()
        pltpu.sync_copy(rows_v, dst.at[pl.ds(base, CHUNK)])

    sbase = wid * SEQ_PER_W

    def seq_step(k, carry):
        do_chunk(tt, sem, msk, seq_out, sbase + k * CHUNK)
        return carry

    lax.fori_loop(0, SEQ_CHUNKS, seq_step, 0)

    fbase = wid * FUT_PER_W
    for k in range(FUT_CHUNKS):
        do_chunk(ttf, semf, None, fut_out, fbase + k * CHUNK)


_sc_lookup = functools.partial(
    pl.kernel,
    out_type=[
        jax.ShapeDtypeStruct((NSEQ, EMB_DIM), jnp.float32),
        jax.ShapeDtypeStruct((NFUT, EMB_DIM), jnp.float32),
    ],
    mesh=plsc.VectorSubcoreMesh(core_axis_name="c", subcore_axis_name="s"),
    scratch_types=[
        pltpu.VMEM((CHUNK,), jnp.int32),
        pltpu.VMEM((CHUNK,), jnp.int32),
        pltpu.VMEM((CHUNK,), jnp.int32),
        pltpu.VMEM((CHUNK,), jnp.int32),
        pltpu.VMEM((CHUNK, EMB_DIM), jnp.float32),
        pltpu.SemaphoreType.DMA,
    ],
)(_body)


def kernel(token_type_ids, sem_ids, seq_mask, sem_ids_fut, token_type_ids_fut, emb):
    tt = token_type_ids.astype(jnp.int32).reshape(NSEQ)
    sm = sem_ids.astype(jnp.int32).reshape(NSEQ)
    mk = seq_mask.astype(jnp.int32).reshape(NSEQ)
    ttf = token_type_ids_fut.astype(jnp.int32).reshape(NFUT)
    smf = sem_ids_fut.astype(jnp.int32).reshape(NFUT)
    seq_flat, fut_flat = _sc_lookup(tt, sm, mk, ttf, smf, emb)
    return (seq_flat.reshape(B, L, EMB_DIM),
            fut_flat.reshape(B, SEM_IDS_DIM, EMB_DIM))


# speedup vs baseline: 49.8002x; 49.8002x over previous
import functools

import jax
import jax.numpy as jnp
from jax import lax
from jax.experimental import pallas as pl
from jax.experimental.pallas import tpu as pltpu
from jax.experimental.pallas import tpu_sc as plsc

NUM_EMB = 1000
SEM_IDS_DIM = 4
EMB_DIM = 128
PAD_IDX = SEM_IDS_DIM * NUM_EMB
B, L = 4096, 200
NSEQ = B * L
NFUT = B * SEM_IDS_DIM
NROWS = NUM_EMB * SEM_IDS_DIM + 1

NC, NS = 2, 16
NW = NC * NS
CHUNK = 256
GATHER = 128
SEQ_PER_W = NSEQ // NW
FUT_PER_W = NFUT // NW
SEQ_CHUNKS = SEQ_PER_W // CHUNK
FUT_CHUNKS = FUT_PER_W // CHUNK
STAGE = 256


def _body(tt, sem, msk, ttf, semf, emb, seq_out, fut_out,
          tt_v, sem_v, msk_v, idx_v, rows_v, emb_sh,
          in_s0, in_s1, g_s0, g_s1, o_s0, o_s1):
    cid = lax.axis_index("c")
    sid = lax.axis_index("s")
    wid = sid * NC + cid
    in_sems = (in_s0, in_s1)
    g_sems = (g_s0, g_s1)
    o_sems = (o_s0, o_s1)

    @pl.when(sid < NS - 1)
    def _():
        stage_v = rows_v.at[0]
        pltpu.sync_copy(emb.at[pl.ds(sid * STAGE, STAGE)], stage_v)
        pltpu.sync_copy(stage_v, emb_sh.at[pl.ds(sid * STAGE, STAGE)])

    @pl.when(sid == NS - 1)
    def _():
        tail = NROWS - (NS - 1) * STAGE
        tail_v = rows_v.at[0, pl.ds(0, tail)]
        pltpu.sync_copy(emb.at[pl.ds((NS - 1) * STAGE, tail)], tail_v)
        pltpu.sync_copy(tail_v, emb_sh.at[pl.ds((NS - 1) * STAGE, tail)])

    plsc.subcore_barrier()

    def issue_in(b, base, masked):
        pltpu.async_copy(tt.at[pl.ds(base, CHUNK)], tt_v.at[b], in_sems[b])
        pltpu.async_copy(sem.at[pl.ds(base, CHUNK)], sem_v.at[b], in_sems[b])
        if masked:
            pltpu.async_copy(msk.at[pl.ds(base, CHUNK)], msk_v.at[b],
                             in_sems[b])

    def wait_in(b, masked):
        pltpu.make_async_copy(tt.at[pl.ds(0, CHUNK)], tt_v.at[b],
                              in_sems[b]).wait()
        pltpu.make_async_copy(sem.at[pl.ds(0, CHUNK)], sem_v.at[b],
                              in_sems[b]).wait()
        if masked:
            pltpu.make_async_copy(msk.at[pl.ds(0, CHUNK)], msk_v.at[b],
                                  in_sems[b]).wait()

    def compute_idx(b, masked):
        for i in range(CHUNK // 16):
            t = tt_v[b, pl.ds(i * 16, 16)]
            s = sem_v[b, pl.ds(i * 16, 16)]
            ids = t * NUM_EMB + s
            if masked:
                m = msk_v[b, pl.ds(i * 16, 16)]
                ids = jnp.where(m != 0, ids, PAD_IDX)
            idx_v[b, pl.ds(i * 16, 16)] = ids

    def gather(b):
        copies = [
            pltpu.async_copy(
                emb_sh.at[idx_v.at[b, pl.ds(j * GATHER, GATHER)]],
                rows_v.at[b, pl.ds(j * GATHER, GATHER)], g_sems[b])
            for j in range(CHUNK // GATHER)
        ]
        for c in copies:
            c.wait()

    def wait_out(b, dst):
        pltpu.make_async_copy(rows_v.at[b], dst.at[pl.ds(0, CHUNK)],
                              o_sems[b]).wait()

    sbase = wid * SEQ_PER_W
    issue_in(0, sbase, True)
    issue_in(1, sbase + CHUNK, True)

    def seq_pair(j, carry):
        for b in (0, 1):
            k = 2 * j + b
            wait_in(b, True)
            compute_idx(b, True)

            @pl.when(k + 2 < SEQ_CHUNKS)
            def _():
                issue_in(b, sbase + (k + 2) * CHUNK, True)

            @pl.when(k >= 2)
            def _():
                wait_out(b, seq_out)

            gather(b)
            pltpu.async_copy(rows_v.at[b],
                             seq_out.at[pl.ds(sbase + k * CHUNK, CHUNK)],
                             o_sems[b])
        return carry

    lax.fori_loop(0, SEQ_CHUNKS // 2, seq_pair, 0)
    wait_out(0, seq_out)
    wait_out(1, seq_out)

    fbase = wid * FUT_PER_W
    for k in range(FUT_CHUNKS):
        base = fbase + k * CHUNK
        pltpu.sync_copy(ttf.at[pl.ds(base, CHUNK)], tt_v.at[0])
        pltpu.sync_copy(semf.at[pl.ds(base, CHUNK)], sem_v.at[0])
        compute_idx(0, False)
        gather(0)
        pltpu.sync_copy(rows_v.at[0], fut_out.at[pl.ds(base, CHUNK)])


_sc_lookup = functools.partial(
    pl.kernel,
    out_type=[
        jax.ShapeDtypeStruct((NSEQ, EMB_DIM), jnp.float32),
        jax.ShapeDtypeStruct((NFUT, EMB_DIM), jnp.float32),
    ],
    mesh=plsc.VectorSubcoreMesh(core_axis_name="c", subcore_axis_name="s"),
    scratch_types=[
        pltpu.VMEM((2, CHUNK), jnp.int32),
        pltpu.VMEM((2, CHUNK), jnp.int32),
        pltpu.VMEM((2, CHUNK), jnp.int32),
        pltpu.VMEM((2, CHUNK), jnp.int32),
        pltpu.VMEM((2, CHUNK, EMB_DIM), jnp.float32),
        pltpu.VMEM_SHARED((NROWS, EMB_DIM), jnp.float32),
        pltpu.SemaphoreType.DMA,
        pltpu.SemaphoreType.DMA,
        pltpu.SemaphoreType.DMA,
        pltpu.SemaphoreType.DMA,
        pltpu.SemaphoreType.DMA,
        pltpu.SemaphoreType.DMA,
    ],
)(_body)


def kernel(token_type_ids, sem_ids, seq_mask, sem_ids_fut, token_type_ids_fut, emb):
    tt = token_type_ids.astype(jnp.int32).reshape(NSEQ)
    sm = sem_ids.astype(jnp.int32).reshape(NSEQ)
    mk = seq_mask.astype(jnp.int32).reshape(NSEQ)
    ttf = token_type_ids_fut.astype(jnp.int32).reshape(NFUT)
    smf = sem_ids_fut.astype(jnp.int32).reshape(NFUT)
    seq_flat, fut_flat = _sc_lookup(tt, sm, mk, ttf, smf, emb)
    return (seq_flat.reshape(B, L, EMB_DIM),
            fut_flat.reshape(B, SEM_IDS_DIM, EMB_DIM))


# speedup vs baseline: 53.0548x; 1.0654x over previous
import functools

import jax
import jax.numpy as jnp
from jax import lax
from jax.experimental import pallas as pl
from jax.experimental.pallas import tpu as pltpu
from jax.experimental.pallas import tpu_sc as plsc

NUM_EMB = 1000
SEM_IDS_DIM = 4
EMB_DIM = 128
PAD_IDX = SEM_IDS_DIM * NUM_EMB
B, L = 4096, 200
NSEQ = B * L
NFUT = B * SEM_IDS_DIM
NROWS = NUM_EMB * SEM_IDS_DIM + 1

NC, NS = 2, 16
NW = NC * NS
CHUNK = 128
NBUF = 5
SEQ_PER_W = NSEQ // NW
FUT_PER_W = NFUT // NW
SEQ_CHUNKS = SEQ_PER_W // CHUNK
FUT_CHUNKS = FUT_PER_W // CHUNK
TRIPS = SEQ_CHUNKS // NBUF
STAGE = 256

assert SEQ_CHUNKS % NBUF == 0 and FUT_CHUNKS <= NBUF


def _body(tt, sem, msk, ttf, semf, emb, seq_out, fut_out, *scratch):
    cid = lax.axis_index("c")
    sid = lax.axis_index("s")
    wid = sid * NC + cid
    tt_v = scratch[0:NBUF]
    sem_v = scratch[NBUF:2 * NBUF]
    msk_v = scratch[2 * NBUF:3 * NBUF]
    idx_v = scratch[3 * NBUF:4 * NBUF]
    rows_v = scratch[4 * NBUF:5 * NBUF]
    emb_sh = scratch[5 * NBUF]
    in_sems = scratch[5 * NBUF + 1:5 * NBUF + 1 + NBUF]
    g_sems = scratch[5 * NBUF + 1 + NBUF:5 * NBUF + 1 + 2 * NBUF]
    o_sems = scratch[5 * NBUF + 1 + 2 * NBUF:5 * NBUF + 1 + 3 * NBUF]

    @pl.when(sid < NS - 1)
    def _():
        for h in range(STAGE // CHUNK):
            base = sid * STAGE + h * CHUNK
            pltpu.sync_copy(emb.at[pl.ds(base, CHUNK)], rows_v[h])
            pltpu.sync_copy(rows_v[h], emb_sh.at[pl.ds(base, CHUNK)])

    @pl.when(sid == NS - 1)
    def _():
        tail = NROWS - (NS - 1) * STAGE
        tbase = (NS - 1) * STAGE
        pltpu.sync_copy(emb.at[pl.ds(tbase, CHUNK)], rows_v[0])
        pltpu.sync_copy(rows_v[0], emb_sh.at[pl.ds(tbase, CHUNK)])
        rest_v = rows_v[1].at[pl.ds(0, tail - CHUNK)]
        pltpu.sync_copy(emb.at[pl.ds(tbase + CHUNK, tail - CHUNK)], rest_v)
        pltpu.sync_copy(rest_v, emb_sh.at[pl.ds(tbase + CHUNK, tail - CHUNK)])

    plsc.subcore_barrier()

    def issue_in(b, base, masked):
        pltpu.async_copy(tt.at[pl.ds(base, CHUNK)], tt_v[b], in_sems[b])
        pltpu.async_copy(sem.at[pl.ds(base, CHUNK)], sem_v[b], in_sems[b])
        if masked:
            pltpu.async_copy(msk.at[pl.ds(base, CHUNK)], msk_v[b],
                             in_sems[b])

    def issue_in_fut(b, base):
        pltpu.async_copy(ttf.at[pl.ds(base, CHUNK)], tt_v[b], in_sems[b])
        pltpu.async_copy(semf.at[pl.ds(base, CHUNK)], sem_v[b], in_sems[b])

    def wait_in(b, masked):
        pltpu.make_async_copy(tt.at[pl.ds(0, CHUNK)], tt_v[b],
                              in_sems[b]).wait()
        pltpu.make_async_copy(sem.at[pl.ds(0, CHUNK)], sem_v[b],
                              in_sems[b]).wait()
        if masked:
            pltpu.make_async_copy(msk.at[pl.ds(0, CHUNK)], msk_v[b],
                                  in_sems[b]).wait()

    def compute_idx(b, masked):
        for i in range(CHUNK // 16):
            t = tt_v[b][pl.ds(i * 16, 16)]
            s = sem_v[b][pl.ds(i * 16, 16)]
            ids = t * NUM_EMB + s
            if masked:
                m = msk_v[b][pl.ds(i * 16, 16)]
                ids = jnp.where(m != 0, ids, PAD_IDX)
            idx_v[b][pl.ds(i * 16, 16)] = ids

    def gather(b):
        pltpu.async_copy(emb_sh.at[idx_v[b]], rows_v[b], g_sems[b]).wait()

    def wait_out(b, dst):
        pltpu.make_async_copy(rows_v[b], dst.at[pl.ds(0, CHUNK)],
                              o_sems[b]).wait()

    sbase = wid * SEQ_PER_W
    for b in range(NBUF):
        issue_in(b, sbase + b * CHUNK, True)

    def seq_trip(j, carry):
        for b in range(NBUF):
            wait_in(b, True)
            compute_idx(b, True)

            @pl.when(j < TRIPS - 1)
            def _():
                issue_in(b, sbase + (NBUF * (j + 1) + b) * CHUNK, True)

            @pl.when(j > 0)
            def _():
                wait_out(b, seq_out)

            gather(b)
            pltpu.async_copy(
                rows_v[b],
                seq_out.at[pl.ds(sbase + (NBUF * j + b) * CHUNK, CHUNK)],
                o_sems[b])
        return carry

    lax.fori_loop(0, TRIPS, seq_trip, 0)

    fbase = wid * FUT_PER_W
    for f in range(FUT_CHUNKS):
        issue_in_fut(f, fbase + f * CHUNK)
    for f in range(FUT_CHUNKS):
        wait_in(f, False)
        compute_idx(f, False)
        wait_out(f, seq_out)
        gather(f)
        pltpu.async_copy(rows_v[f],
                         fut_out.at[pl.ds(fbase + f * CHUNK, CHUNK)],
                         o_sems[f])
    for f in range(FUT_CHUNKS):
        wait_out(f, fut_out)
    for b in range(FUT_CHUNKS, NBUF):
        wait_out(b, seq_out)


_sc_lookup = functools.partial(
    pl.kernel,
    out_type=[
        jax.ShapeDtypeStruct((NSEQ, EMB_DIM), jnp.float32),
        jax.ShapeDtypeStruct((NFUT, EMB_DIM), jnp.float32),
    ],
    mesh=plsc.VectorSubcoreMesh(core_axis_name="c", subcore_axis_name="s"),
    scratch_types=(
        [pltpu.VMEM((CHUNK,), jnp.int32)] * (4 * NBUF)
        + [pltpu.VMEM((CHUNK, EMB_DIM), jnp.float32)] * NBUF
        + [pltpu.VMEM_SHARED((NROWS, EMB_DIM), jnp.float32)]
        + [pltpu.SemaphoreType.DMA] * (3 * NBUF)),
)(_body)


def kernel(token_type_ids, sem_ids, seq_mask, sem_ids_fut, token_type_ids_fut, emb):
    tt = token_type_ids.astype(jnp.int32).reshape(NSEQ)
    sm = sem_ids.astype(jnp.int32).reshape(NSEQ)
    mk = seq_mask.astype(jnp.int32).reshape(NSEQ)
    ttf = token_type_ids_fut.astype(jnp.int32).reshape(NFUT)
    smf = sem_ids_fut.astype(jnp.int32).reshape(NFUT)
    seq_flat, fut_flat = _sc_lookup(tt, sm, mk, ttf, smf, emb)
    return (seq_flat.reshape(B, L, EMB_DIM),
            fut_flat.reshape(B, SEM_IDS_DIM, EMB_DIM))
